# baseline (speedup 1.0000x reference)
import functools

import jax
import jax.numpy as jnp
from jax import lax
from jax.experimental import pallas as pl
from jax.experimental.pallas import tpu as pltpu
from jax.experimental.pallas import tpu_sc as plsc

N_MASK = 1024
BS = 64
N = 8192
LANES = 16
NVEC = N // LANES
ROWS_PER_W = 2


def _loss_kernel(out_ref, tgt_ref, loss_ref):
    x = out_ref[...]
    t = tgt_ref[...]
    loss_ref[...] = jnp.maximum(x, 0.0) - x * t + jnp.log1p(jnp.exp(-jnp.abs(x)))


def _tc_loss(output, target):
    return pl.pallas_call(
        _loss_kernel,
        out_shape=jax.ShapeDtypeStruct((BS, N), jnp.float32),
    )(output, target)


def _sc_topk_body(loss_hbm, out_hbm, row_v, out_v):
    wid = lax.axis_index("s") * 2 + lax.axis_index("c")
    lane = lax.broadcasted_iota(jnp.int32, (LANES,), 0)
    out_vec = jnp.zeros((LANES,), jnp.float32)

    for r in range(ROWS_PER_W):
        row = wid * ROWS_PER_W + r
        pltpu.sync_copy(loss_hbm.at[row], row_v)

        def bitstep(i, cur):
            trial = cur | (jnp.int32(1) << (jnp.int32(30) - i))

            def cnt_body(j, acc):
                v = lax.bitcast_convert_type(row_v[pl.ds(j * LANES, LANES)], jnp.int32)
                return acc + jnp.where(v >= trial, jnp.int32(1), jnp.int32(0))

            acc = lax.fori_loop(0, NVEC, cnt_body, jnp.zeros((LANES,), jnp.int32))
            return jnp.where(jnp.sum(acc) >= N_MASK, trial, cur)

        kth = lax.fori_loop(0, 31, bitstep, jnp.int32(0))

        def sum_body(j, carry):
            sv, cv = carry
            v = row_v[pl.ds(j * LANES, LANES)]
            m = lax.bitcast_convert_type(v, jnp.int32) > kth
            sv = sv + jnp.where(m, v, 0.0)
            cv = cv + jnp.where(m, jnp.int32(1), jnp.int32(0))
            return sv, cv

        sv, cv = lax.fori_loop(
            0, NVEC, sum_body,
            (jnp.zeros((LANES,), jnp.float32), jnp.zeros((LANES,), jnp.int32)),
        )
        thr_v = lax.bitcast_convert_type(jnp.full((LANES,), kth, jnp.int32), jnp.float32)
        n_tie = (jnp.int32(N_MASK) - jnp.sum(cv)).astype(jnp.float32)
        total_v = jnp.sum(sv) + n_tie * thr_v
        out_vec = jnp.where(lane == r, total_v, out_vec)

    out_v[...] = out_vec
    pltpu.sync_copy(out_v, out_hbm.at[wid])


@functools.partial(
    pl.kernel,
    out_type=jax.ShapeDtypeStruct((BS // ROWS_PER_W, LANES), jnp.float32),
    mesh=plsc.VectorSubcoreMesh(core_axis_name="c", subcore_axis_name="s"),
    compiler_params=pltpu.CompilerParams(needs_layout_passes=False),
    scratch_types=[
        pltpu.VMEM((N,), jnp.float32),
        pltpu.VMEM((LANES,), jnp.float32),
    ],
)
def _sc_topk(loss_hbm, out_hbm, row_v, out_v):
    _sc_topk_body(loss_hbm, out_hbm, row_v, out_v)


@jax.jit
def kernel(output, target):
    loss = _tc_loss(output, target)
    row_sums = _sc_topk(loss)
    return (jnp.sum(row_sums) / (BS * N_MASK)).astype(jnp.float32)


# speedup vs baseline: 2.9680x; 2.9680x over previous
import functools

import jax
import jax.numpy as jnp
from jax import lax
from jax.experimental import pallas as pl
from jax.experimental.pallas import tpu as pltpu
from jax.experimental.pallas import tpu_sc as plsc

N_MASK = 1024
BS = 64
N = 8192
LANES = 16
NVEC = N // LANES
ROWS_PER_W = 2

_LEVELS = ((22, 512, 31), (15, 128, 22), (8, 128, 15), (0, 256, 8))


def _loss_kernel(out_ref, tgt_ref, loss_ref):
    x = out_ref[...]
    t = tgt_ref[...]
    loss_ref[...] = jnp.maximum(x, 0.0) - x * t + jnp.log1p(jnp.exp(-jnp.abs(x)))


def _tc_loss(output, target):
    return pl.pallas_call(
        _loss_kernel,
        out_shape=jax.ShapeDtypeStruct((BS, N), jnp.float32),
    )(output, target)


def _sc_topk_body(loss_hbm, zeros_hbm, out_hbm, row_v, hist_v, out_v):
    wid = lax.axis_index("s") * 2 + lax.axis_index("c")
    lane = lax.broadcasted_iota(jnp.int32, (LANES,), 0)
    ones = jnp.ones((LANES,), jnp.int32)
    out_vec = jnp.zeros((LANES,), jnp.float32)

    for r in range(ROWS_PER_W):
        row = wid * ROWS_PER_W + r
        pltpu.sync_copy(loss_hbm.at[row], row_v)

        base = jnp.int32(0)
        rank = jnp.int32(N_MASK)

        for shift, nbins, ushift in _LEVELS:
            nwords = nbins * LANES
            pltpu.sync_copy(zeros_hbm.at[pl.ds(0, nwords)],
                            hist_v.at[pl.ds(0, nwords)])
            pref = base >> ushift
            mbits = jnp.int32(nbins - 1)

            @plsc.parallel_loop(0, NVEC, unroll=8)
            def _(j, shift=shift, ushift=ushift, pref=pref, mbits=mbits):
                v = row_v[pl.ds(j * LANES, LANES)]
                bits = lax.bitcast_convert_type(v, jnp.int32)
                b = (bits >> shift) & mbits
                m = (bits >> ushift) == pref
                plsc.addupdate_scatter(hist_v, [b * LANES + lane], ones, mask=m)

            nch = nbins // LANES

            def sbody(i, carry, nch=nch):
                B, cnt_ge, MB, seen, found = carry
                c = nch - 1 - i
                t = jnp.zeros((LANES,), jnp.int32)
                for l in range(LANES):
                    t = t + plsc.load_gather(hist_v, [(c * LANES + lane) * LANES + l])
                pre = plsc.cumsum(t)
                tot = jnp.sum(t)
                suffix = seen + tot - pre + t
                m = suffix >= rank
                nm = jnp.sum(m.astype(jnp.int32))
                hit = (found == 0) & (nm > 0)
                j = nm - 1
                B = jnp.where(hit, c * LANES + j, B)
                cnt_ge = jnp.where(hit, jnp.sum(jnp.where(lane == j, suffix, 0)), cnt_ge)
                MB = jnp.where(hit, jnp.sum(jnp.where(lane == j, t, 0)), MB)
                return B, cnt_ge, MB, seen + tot, found | (nm > 0).astype(jnp.int32)

            B, cnt_ge, MB, _, _ = lax.fori_loop(
                0, nch, sbody, (jnp.int32(0),) * 5)
            rank = rank - (cnt_ge - MB)
            base = base | (B << shift)

        @plsc.parallel_loop(0, NVEC, unroll=8,
                            carry=(jnp.zeros((LANES,), jnp.float32),
                                   jnp.zeros((LANES,), jnp.int32)))
        def final_carry(j, carry, base=base):
            sv, cv = carry
            v = row_v[pl.ds(j * LANES, LANES)]
            m = lax.bitcast_convert_type(v, jnp.int32) > base
            return (sv + jnp.where(m, v, 0.0),
                    cv + jnp.where(m, jnp.int32(1), jnp.int32(0)))

        sv, cv = final_carry
        thr_v = lax.bitcast_convert_type(jnp.full((LANES,), base, jnp.int32),
                                         jnp.float32)
        n_tie = (jnp.int32(N_MASK) - jnp.sum(cv)).astype(jnp.float32)
        total_v = jnp.sum(sv) + n_tie * thr_v
        out_vec = jnp.where(lane == r, total_v, out_vec)

    out_v[...] = out_vec
    pltpu.sync_copy(out_v, out_hbm.at[wid])


@functools.partial(
    pl.kernel,
    out_type=jax.ShapeDtypeStruct((BS // ROWS_PER_W, LANES), jnp.float32),
    mesh=plsc.VectorSubcoreMesh(core_axis_name="c", subcore_axis_name="s"),
    compiler_params=pltpu.CompilerParams(needs_layout_passes=False),
    scratch_types=[
        pltpu.VMEM((N,), jnp.float32),
        pltpu.VMEM((N,), jnp.int32),
        pltpu.VMEM((LANES,), jnp.float32),
    ],
)
def _sc_topk(loss_hbm, zeros_hbm, out_hbm, row_v, hist_v, out_v):
    _sc_topk_body(loss_hbm, zeros_hbm, out_hbm, row_v, hist_v, out_v)


@jax.jit
def kernel(output, target):
    loss = _tc_loss(output, target)
    zeros = jnp.zeros((N,), jnp.int32)
    row_sums = _sc_topk(loss, zeros)
    return (jnp.sum(row_sums) / (BS * N_MASK)).astype(jnp.float32)


# speedup vs baseline: 3.1662x; 1.0668x over previous
import functools

import jax
import jax.numpy as jnp
from jax import lax
from jax.experimental import pallas as pl
from jax.experimental.pallas import tpu as pltpu
from jax.experimental.pallas import tpu_sc as plsc

N_MASK = 1024
BS = 64
N = 8192
LANES = 16
NVEC = N // LANES
ROWS_PER_W = 2

_LEVELS = ((22, 512, 31), (15, 128, 22), (8, 128, 15), (0, 256, 8))


def _loss_kernel(out_ref, tgt_ref, loss_ref):
    x = out_ref[...]
    t = tgt_ref[...]
    loss_ref[...] = jnp.maximum(x, 0.0) - x * t + jnp.log1p(jnp.exp(-jnp.abs(x)))


def _tc_loss(output, target):
    return pl.pallas_call(
        _loss_kernel,
        out_shape=jax.ShapeDtypeStruct((BS, N), jnp.float32),
    )(output, target)


def _find_kth_bits(row_v, hist_v, tot_v, lane, ones):
    base = jnp.int32(0)
    rank = jnp.int32(N_MASK)
    zz = jnp.zeros((LANES,), jnp.int32)

    for shift, nbins, ushift in _LEVELS:
        nch = nbins // LANES

        @plsc.parallel_loop(0, nch * LANES, unroll=8)
        def _(j):
            hist_v[pl.ds(j * LANES, LANES)] = zz

        pref = base >> ushift
        mbits = jnp.int32(nbins - 1)

        @plsc.parallel_loop(0, NVEC, unroll=8)
        def _(j, shift=shift, ushift=ushift, pref=pref, mbits=mbits,
              nbins=nbins):
            v = row_v[pl.ds(j * LANES, LANES)]
            bits = lax.bitcast_convert_type(v, jnp.int32)
            b = (bits >> shift) & mbits
            m = (bits >> ushift) == pref
            plsc.addupdate_scatter(hist_v, [lane * nbins + b], ones, mask=m)

        @plsc.parallel_loop(0, nch, unroll=2)
        def _(c, nbins=nbins):
            parts = [hist_v[pl.ds(l * nbins + c * LANES, LANES)]
                     for l in range(LANES)]
            while len(parts) > 1:
                parts = [parts[i] + parts[i + 1]
                         for i in range(0, len(parts), 2)]
            tot_v[pl.ds(c * LANES, LANES)] = parts[0]

        def sbody(i, carry, nch=nch, rank=rank):
            seen, B, cnt_ge, MB, found = carry
            c = nch - 1 - i
            t = tot_v[pl.ds(c * LANES, LANES)]
            tot = jnp.sum(t)
            hit = (found == 0) & (seen + tot >= rank)

            def on_hit(_):
                pre = plsc.cumsum(t)
                suffix = seen + tot - pre + t
                m = suffix >= rank
                j = jnp.sum(m.astype(jnp.int32)) - 1
                return (c * LANES + j,
                        jnp.sum(jnp.where(lane == j, suffix, 0)),
                        jnp.sum(jnp.where(lane == j, t, 0)))

            B, cnt_ge, MB = lax.cond(hit, on_hit,
                                     lambda _: (B, cnt_ge, MB), 0)
            return (seen + tot, B, cnt_ge, MB,
                    found | hit.astype(jnp.int32))

        _, B, cnt_ge, MB, _ = lax.fori_loop(0, nch, sbody,
                                            (jnp.int32(0),) * 5)
        rank = rank - (cnt_ge - MB)
        base = base | (B << shift)
    return base


def _sc_topk_body(loss_hbm, out_hbm, row_v, hist_v, tot_v, out_v):
    wid = lax.axis_index("s") * 2 + lax.axis_index("c")
    lane = lax.broadcasted_iota(jnp.int32, (LANES,), 0)
    ones = jnp.ones((LANES,), jnp.int32)
    out_vec = jnp.zeros((LANES,), jnp.float32)

    for r in range(ROWS_PER_W):
        row = wid * ROWS_PER_W + r
        pltpu.sync_copy(loss_hbm.at[row], row_v)

        kth = _find_kth_bits(row_v, hist_v, tot_v, lane, ones)

        @plsc.parallel_loop(0, NVEC, unroll=8,
                            carry=(jnp.zeros((LANES,), jnp.float32),
                                   jnp.zeros((LANES,), jnp.int32)))
        def final_carry(j, carry, kth=kth):
            sv, cv = carry
            v = row_v[pl.ds(j * LANES, LANES)]
            m = lax.bitcast_convert_type(v, jnp.int32) > kth
            return (sv + jnp.where(m, v, 0.0),
                    cv + jnp.where(m, jnp.int32(1), jnp.int32(0)))

        sv, cv = final_carry
        thr_v = lax.bitcast_convert_type(jnp.full((LANES,), kth, jnp.int32),
                                         jnp.float32)
        n_tie = (jnp.int32(N_MASK) - jnp.sum(cv)).astype(jnp.float32)
        total_v = jnp.sum(sv) + n_tie * thr_v
        out_vec = jnp.where(lane == r, total_v, out_vec)

    out_v[...] = out_vec
    pltpu.sync_copy(out_v, out_hbm.at[wid])


@functools.partial(
    pl.kernel,
    out_type=jax.ShapeDtypeStruct((BS // ROWS_PER_W, LANES), jnp.float32),
    mesh=plsc.VectorSubcoreMesh(core_axis_name="c", subcore_axis_name="s"),
    compiler_params=pltpu.CompilerParams(needs_layout_passes=False),
    scratch_types=[
        pltpu.VMEM((N,), jnp.float32),
        pltpu.VMEM((N,), jnp.int32),
        pltpu.VMEM((512,), jnp.int32),
        pltpu.VMEM((LANES,), jnp.float32),
    ],
)
def _sc_topk(loss_hbm, out_hbm, row_v, hist_v, tot_v, out_v):
    _sc_topk_body(loss_hbm, out_hbm, row_v, hist_v, tot_v, out_v)


@jax.jit
def kernel(output, target):
    loss = _tc_loss(output, target)
    row_sums = _sc_topk(loss)
    return (jnp.sum(row_sums) / (BS * N_MASK)).astype(jnp.float32)


# speedup vs baseline: 3.4237x; 1.0813x over previous
import functools

import jax
import jax.numpy as jnp
from jax import lax
from jax.experimental import pallas as pl
from jax.experimental.pallas import tpu as pltpu
from jax.experimental.pallas import tpu_sc as plsc

N_MASK = 1024
BS = 64
N = 8192
LANES = 16
NVEC = N // LANES
ROWS_PER_W = 2

_LEVELS = ((22, 512, 31), (15, 128, 22), (8, 128, 15), (0, 256, 8))


def _loss_kernel(out_ref, tgt_ref, loss_ref):
    x = out_ref[...]
    t = tgt_ref[...]
    loss_ref[...] = jnp.maximum(x, 0.0) - x * t + jnp.log1p(jnp.exp(-jnp.abs(x)))


def _tc_loss(output, target):
    return pl.pallas_call(
        _loss_kernel,
        out_shape=jax.ShapeDtypeStruct((BS, N), jnp.float32),
    )(output, target)


def _find_kth_bits(row_v, hist_v, tot_v, lane, ones):
    base = jnp.int32(0)
    rank = jnp.int32(N_MASK)
    zz = jnp.zeros((LANES,), jnp.int32)

    for shift, nbins, ushift in _LEVELS:
        nch = nbins // LANES

        @plsc.parallel_loop(0, nch * LANES, unroll=8)
        def _(j):
            hist_v[pl.ds(j * LANES, LANES)] = zz

        pref = base >> ushift
        mbits = jnp.int32(nbins - 1)

        @plsc.parallel_loop(0, NVEC, unroll=8)
        def _(j, shift=shift, ushift=ushift, pref=pref, mbits=mbits,
              nbins=nbins):
            v = row_v[pl.ds(j * LANES, LANES)]
            bits = lax.bitcast_convert_type(v, jnp.int32)
            b = (bits >> shift) & mbits
            m = (bits >> ushift) == pref
            plsc.addupdate_scatter(hist_v, [lane * nbins + b], ones, mask=m)

        @plsc.parallel_loop(0, nch, unroll=2)
        def _(c, nbins=nbins):
            parts = [hist_v[pl.ds(l * nbins + c * LANES, LANES)]
                     for l in range(LANES)]
            while len(parts) > 1:
                parts = [parts[i] + parts[i + 1]
                         for i in range(0, len(parts), 2)]
            tot_v[pl.ds(c * LANES, LANES)] = parts[0]

        ngroups = (nch + 15) // LANES
        seen = jnp.int32(0)
        found = jnp.int32(0)
        hc_v = jnp.zeros((LANES,), jnp.int32)
        A_v = jnp.zeros((LANES,), jnp.int32)
        tc_v = jnp.zeros((LANES,), jnp.int32)
        for g in range(ngroups - 1, -1, -1):
            mv = (g * LANES + lane) < nch
            cs = jnp.zeros((LANES,), jnp.int32)
            for w in range(LANES):
                gg = plsc.load_gather(
                    tot_v, [(g * LANES + lane) * LANES + w], mask=mv)
                cs = cs + jnp.where(mv, gg, 0)
            pre = plsc.cumsum(cs)
            tg = jnp.sum(jnp.where(lane == LANES - 1, pre, 0))
            suffix = (seen + tg) - pre + cs
            m = (suffix >= rank) & mv
            pc = plsc.all_reduce_population_count(m)
            j_v = pc - 1
            hit = (found == 0) & (jnp.sum(jnp.where(lane == 0, pc, 0)) > 0)
            hit_i = hit.astype(jnp.int32)
            sel = jnp.where(lane == j_v, hit_i, 0)
            hc_v = hc_v + sel * (g * LANES + j_v)
            A_v = A_v + sel * (suffix - cs)
            tc_v = tc_v + sel * cs
            found = found + hit_i
            seen = seen + tg
        hc = jnp.sum(hc_v)
        A = jnp.sum(A_v)
        tot_c = jnp.sum(tc_v)

        t = tot_v[pl.ds(hc * LANES, LANES)]
        pre2 = plsc.cumsum(t)
        suffix3 = (A + tot_c) - pre2 + t
        m2 = suffix3 >= rank
        j2 = plsc.all_reduce_population_count(m2) - 1
        sel2 = jnp.where(lane == j2, 1, 0)
        B = jnp.sum(sel2 * (hc * LANES + j2))
        cnt_ge = jnp.sum(sel2 * suffix3)
        MB = jnp.sum(sel2 * t)

        rank = rank - (cnt_ge - MB)
        base = base | (B << shift)
    return base


def _row_sum(row_v, hist_v, tot_v, lane, ones):
    kth = _find_kth_bits(row_v, hist_v, tot_v, lane, ones)

    @plsc.parallel_loop(0, NVEC, unroll=8,
                        carry=(jnp.zeros((LANES,), jnp.float32),
                               jnp.zeros((LANES,), jnp.int32)))
    def final_carry(j, carry, kth=kth):
        sv, cv = carry
        v = row_v[pl.ds(j * LANES, LANES)]
        m = lax.bitcast_convert_type(v, jnp.int32) > kth
        return (sv + jnp.where(m, v, 0.0),
                cv + jnp.where(m, jnp.int32(1), jnp.int32(0)))

    sv, cv = final_carry
    thr_v = lax.bitcast_convert_type(jnp.full((LANES,), kth, jnp.int32),
                                     jnp.float32)
    n_tie = (jnp.int32(N_MASK) - jnp.sum(cv)).astype(jnp.float32)
    return jnp.sum(sv) + n_tie * thr_v


def _sc_topk_body(loss_hbm, out_hbm, row0_v, row1_v, hist_v, tot_v, out_v,
                  sem0, sem1):
    wid = lax.axis_index("s") * 2 + lax.axis_index("c")
    lane = lax.broadcasted_iota(jnp.int32, (LANES,), 0)
    ones = jnp.ones((LANES,), jnp.int32)

    row = wid * ROWS_PER_W
    cp0 = pltpu.async_copy(loss_hbm.at[row], row0_v, sem0)
    cp1 = pltpu.async_copy(loss_hbm.at[row + 1], row1_v, sem1)

    cp0.wait()
    t0 = _row_sum(row0_v, hist_v, tot_v, lane, ones)
    cp1.wait()
    t1 = _row_sum(row1_v, hist_v, tot_v, lane, ones)

    out_v[...] = jnp.where(lane == 0, t0, jnp.where(lane == 1, t1, 0.0))
    pltpu.sync_copy(out_v, out_hbm.at[wid])


@functools.partial(
    pl.kernel,
    out_type=jax.ShapeDtypeStruct((BS // ROWS_PER_W, LANES), jnp.float32),
    mesh=plsc.VectorSubcoreMesh(core_axis_name="c", subcore_axis_name="s"),
    compiler_params=pltpu.CompilerParams(needs_layout_passes=False),
    scratch_types=[
        pltpu.VMEM((N,), jnp.float32),
        pltpu.VMEM((N,), jnp.float32),
        pltpu.VMEM((N,), jnp.int32),
        pltpu.VMEM((512,), jnp.int32),
        pltpu.VMEM((LANES,), jnp.float32),
        pltpu.SemaphoreType.DMA,
        pltpu.SemaphoreType.DMA,
    ],
)
def _sc_topk(loss_hbm, out_hbm, row0_v, row1_v, hist_v, tot_v, out_v,
             sem0, sem1):
    _sc_topk_body(loss_hbm, out_hbm, row0_v, row1_v, hist_v, tot_v, out_v,
                  sem0, sem1)


@jax.jit
def kernel(output, target):
    loss = _tc_loss(output, target)
    row_sums = _sc_topk(loss)
    return (jnp.sum(row_sums) / (BS * N_MASK)).astype(jnp.float32)
